# Initial kernel scaffold; baseline (speedup 1.0000x reference)
import jax
import jax.numpy as jnp
from jax.experimental import pallas as pl


def kernel(hidden_states, weight):
    raise NotImplementedError("write your pallas kernel here")


# speedup vs baseline: 2.5600x; 2.5600x over previous
import functools

import jax
import jax.numpy as jnp
from jax import lax
from jax.experimental import pallas as pl
from jax.experimental.pallas import tpu as pltpu

_TOP_K = 8
_E = 64
_ALPHA = 0.1


def _router_kernel(hs_ref, w_ref, idx_ref, wt_ref, ce_ref, ss_ref, aux_ref,
                   *, blk, nsteps, blocks_per_batch, bsz, seq_len):
    i = pl.program_id(0)

    @pl.when(i == 0)
    def _init():
        ce_ref[:, :] = jnp.zeros_like(ce_ref)
        ss_ref[:, :] = jnp.zeros_like(ss_ref)

    logits = lax.dot_general(
        hs_ref[:, :], w_ref[:, :], (((1,), (1,)), ((), ())),
        preferred_element_type=jnp.float32, precision=lax.Precision.DEFAULT)

    m = jnp.max(logits, axis=1, keepdims=True)
    e = jnp.exp(logits - m)
    s = jnp.sum(e, axis=1, keepdims=True)
    scores = e / s

    cols = lax.broadcasted_iota(jnp.int32, (blk, _E), 1)
    work = scores
    sel = jnp.zeros((blk, _E), jnp.float32)
    vals, idxs = [], []
    for _ in range(_TOP_K):
        mx = jnp.max(work, axis=1, keepdims=True)
        is_mx = work == mx
        pick = jnp.min(jnp.where(is_mx, cols, _E), axis=1, keepdims=True)
        onehot = cols == pick
        vals.append(mx)
        idxs.append(pick)
        sel = sel + onehot.astype(jnp.float32)
        work = jnp.where(onehot, -jnp.inf, work)

    topw = jnp.concatenate(vals, axis=1)
    topi = jnp.concatenate(idxs, axis=1)
    denom = jnp.sum(topw, axis=1, keepdims=True) + 1e-20
    wt_ref[:, :] = topw / denom
    idx_ref[:, :] = topi

    counts = jnp.sum(sel, axis=0, keepdims=True)
    ssum = jnp.sum(scores, axis=0, keepdims=True)
    b = i // blocks_per_batch
    brow = lax.broadcasted_iota(jnp.int32, (bsz, 1), 0)
    bmask = (brow == b).astype(jnp.float32)
    ce_ref[:, :] += bmask * counts
    ss_ref[:, :] += bmask * ssum

    @pl.when(i == nsteps - 1)
    def _fin():
        ce = ce_ref[:, :] * (_E / (seq_len * _TOP_K))
        ms = ss_ref[:, :] / seq_len
        aux_ref[:, :] = jnp.sum(ce * ms, keepdims=True).reshape(1, 1) * (_ALPHA / bsz)


def kernel(hidden_states, weight):
    bsz, seq_len, hid = hidden_states.shape
    n_tok = bsz * seq_len
    blk = 512
    nsteps = n_tok // blk
    hs = hidden_states.reshape(n_tok, hid)

    out_shapes = (
        jax.ShapeDtypeStruct((n_tok, _TOP_K), jnp.int32),
        jax.ShapeDtypeStruct((n_tok, _TOP_K), jnp.float32),
        jax.ShapeDtypeStruct((bsz, _E), jnp.float32),
        jax.ShapeDtypeStruct((bsz, _E), jnp.float32),
        jax.ShapeDtypeStruct((1, 1), jnp.float32),
    )
    in_specs = [
        pl.BlockSpec((blk, hid), lambda i: (i, 0)),
        pl.BlockSpec((_E, hid), lambda i: (0, 0)),
    ]
    out_specs = (
        pl.BlockSpec((blk, _TOP_K), lambda i: (i, 0)),
        pl.BlockSpec((blk, _TOP_K), lambda i: (i, 0)),
        pl.BlockSpec((bsz, _E), lambda i: (0, 0)),
        pl.BlockSpec((bsz, _E), lambda i: (0, 0)),
        pl.BlockSpec((1, 1), lambda i: (0, 0)),
    )
    idx, wt, _ce, _ss, aux = pl.pallas_call(
        functools.partial(
            _router_kernel, blk=blk, nsteps=nsteps,
            blocks_per_batch=seq_len // blk, bsz=bsz, seq_len=seq_len),
        grid=(nsteps,),
        in_specs=in_specs,
        out_specs=out_specs,
        out_shape=out_shapes,
        compiler_params=pltpu.CompilerParams(
            dimension_semantics=("arbitrary",)),
    )(hs, weight)
    idx = idx.reshape(n_tok, _TOP_K)
    wt = wt.reshape(n_tok, _TOP_K)
    return idx, wt, aux[0, 0]


# speedup vs baseline: 2.9293x; 1.1443x over previous
import functools

import jax
import jax.numpy as jnp
from jax import lax
from jax.experimental import pallas as pl
from jax.experimental.pallas import tpu as pltpu

_TOP_K = 8
_E = 64
_ALPHA = 0.1


def _router_kernel(hs_ref, w_ref, idx_ref, wt_ref, ce_ref, ss_ref, aux_ref,
                   *, blk, nsteps, blocks_per_batch, bsz, seq_len):
    i = pl.program_id(0)

    @pl.when(i == 0)
    def _init():
        ce_ref[:, :] = jnp.zeros_like(ce_ref)
        ss_ref[:, :] = jnp.zeros_like(ss_ref)

    logits = lax.dot_general(
        hs_ref[:, :], w_ref[:, :], (((1,), (1,)), ((), ())),
        preferred_element_type=jnp.float32, precision=lax.Precision.DEFAULT)

    m = jnp.max(logits, axis=1, keepdims=True)
    e = jnp.exp(logits - m)
    s = jnp.sum(e, axis=1, keepdims=True)
    scores = e / s

    cols = lax.broadcasted_iota(jnp.int32, (blk, _E), 1)
    work = scores
    vals, idxs = [], []
    for _ in range(_TOP_K):
        mx = jnp.max(work, axis=1, keepdims=True)
        pick = jnp.argmax(work, axis=1).reshape(blk, 1).astype(jnp.int32)
        vals.append(mx)
        idxs.append(pick)
        work = jnp.where(cols == pick, -jnp.inf, work)

    topw = jnp.concatenate(vals, axis=1)
    topi = jnp.concatenate(idxs, axis=1)
    denom = jnp.sum(topw, axis=1, keepdims=True) + 1e-20
    wt_ref[:, :] = topw / denom
    idx_ref[:, :] = topi

    sel = (work == -jnp.inf).astype(jnp.float32)
    counts = jnp.sum(sel, axis=0, keepdims=True)
    ssum = jnp.sum(scores, axis=0, keepdims=True)
    b = i // blocks_per_batch
    brow = lax.broadcasted_iota(jnp.int32, (bsz, 1), 0)
    bmask = (brow == b).astype(jnp.float32)
    ce_ref[:, :] += bmask * counts
    ss_ref[:, :] += bmask * ssum

    @pl.when(i == nsteps - 1)
    def _fin():
        ce = ce_ref[:, :] * (_E / (seq_len * _TOP_K))
        ms = ss_ref[:, :] / seq_len
        aux_ref[:, :] = jnp.sum(ce * ms, keepdims=True).reshape(1, 1) * (_ALPHA / bsz)


def kernel(hidden_states, weight):
    bsz, seq_len, hid = hidden_states.shape
    n_tok = bsz * seq_len
    blk = 512
    nsteps = n_tok // blk
    hs = hidden_states.reshape(n_tok, hid)

    out_shapes = (
        jax.ShapeDtypeStruct((n_tok, _TOP_K), jnp.int32),
        jax.ShapeDtypeStruct((n_tok, _TOP_K), jnp.float32),
        jax.ShapeDtypeStruct((bsz, _E), jnp.float32),
        jax.ShapeDtypeStruct((bsz, _E), jnp.float32),
        jax.ShapeDtypeStruct((1, 1), jnp.float32),
    )
    in_specs = [
        pl.BlockSpec((blk, hid), lambda i: (i, 0)),
        pl.BlockSpec((_E, hid), lambda i: (0, 0)),
    ]
    out_specs = (
        pl.BlockSpec((blk, _TOP_K), lambda i: (i, 0)),
        pl.BlockSpec((blk, _TOP_K), lambda i: (i, 0)),
        pl.BlockSpec((bsz, _E), lambda i: (0, 0)),
        pl.BlockSpec((bsz, _E), lambda i: (0, 0)),
        pl.BlockSpec((1, 1), lambda i: (0, 0)),
    )
    idx, wt, _ce, _ss, aux = pl.pallas_call(
        functools.partial(
            _router_kernel, blk=blk, nsteps=nsteps,
            blocks_per_batch=seq_len // blk, bsz=bsz, seq_len=seq_len),
        grid=(nsteps,),
        in_specs=in_specs,
        out_specs=out_specs,
        out_shape=out_shapes,
        compiler_params=pltpu.CompilerParams(
            dimension_semantics=("arbitrary",)),
    )(hs, weight)
    idx = idx.reshape(n_tok, _TOP_K)
    wt = wt.reshape(n_tok, _TOP_K)
    return idx, wt, aux[0, 0]


# speedup vs baseline: 3.1189x; 1.0647x over previous
import functools

import jax
import jax.numpy as jnp
from jax import lax
from jax.experimental import pallas as pl
from jax.experimental.pallas import tpu as pltpu

_TOP_K = 8
_E = 64
_ALPHA = 0.1


def _router_kernel(hs_ref, w_ref, idx_ref, wt_ref, ce_ref, ss_ref, aux_ref,
                   *, blk, nsteps, blocks_per_batch, bsz, seq_len):
    i = pl.program_id(0)

    @pl.when(i == 0)
    def _init():
        ce_ref[:, :] = jnp.zeros_like(ce_ref)
        ss_ref[:, :] = jnp.zeros_like(ss_ref)

    logits = lax.dot_general(
        hs_ref[:, :], w_ref[:, :], (((1,), (1,)), ((), ())),
        preferred_element_type=jnp.float32, precision=lax.Precision.DEFAULT)

    m = jnp.max(logits, axis=1, keepdims=True)
    e = jnp.exp(logits - m)
    s = jnp.sum(e, axis=1, keepdims=True)
    scores = e / s

    cols = lax.broadcasted_iota(jnp.int32, (blk, _E), 1)
    work = scores
    vals, idxs = [], []
    for _ in range(_TOP_K):
        mx = jnp.max(work, axis=1, keepdims=True)
        pick = jnp.argmax(work, axis=1).reshape(blk, 1).astype(jnp.int32)
        vals.append(mx)
        idxs.append(pick)
        work = jnp.where(cols == pick, -jnp.inf, work)

    topw = jnp.concatenate(vals, axis=1)
    topi = jnp.concatenate(idxs, axis=1)
    denom = jnp.sum(topw, axis=1, keepdims=True) + 1e-20
    wt_ref[:, :] = topw / denom
    idx_ref[:, :] = topi

    sel = (work == -jnp.inf).astype(jnp.float32)
    counts = jnp.sum(sel, axis=0, keepdims=True)
    ssum = jnp.sum(scores, axis=0, keepdims=True)
    b = i // blocks_per_batch
    brow = lax.broadcasted_iota(jnp.int32, (bsz, 1), 0)
    bmask = (brow == b).astype(jnp.float32)
    ce_ref[:, :] += bmask * counts
    ss_ref[:, :] += bmask * ssum

    @pl.when(i == nsteps - 1)
    def _fin():
        ce = ce_ref[:, :] * (_E / (seq_len * _TOP_K))
        ms = ss_ref[:, :] / seq_len
        aux_ref[:, :] = jnp.sum(ce * ms, keepdims=True).reshape(1, 1) * (_ALPHA / bsz)


def kernel(hidden_states, weight):
    bsz, seq_len, hid = hidden_states.shape
    n_tok = bsz * seq_len
    blk = 1024
    nsteps = n_tok // blk
    hs = hidden_states.reshape(n_tok, hid)

    out_shapes = (
        jax.ShapeDtypeStruct((n_tok, _TOP_K), jnp.int32),
        jax.ShapeDtypeStruct((n_tok, _TOP_K), jnp.float32),
        jax.ShapeDtypeStruct((bsz, _E), jnp.float32),
        jax.ShapeDtypeStruct((bsz, _E), jnp.float32),
        jax.ShapeDtypeStruct((1, 1), jnp.float32),
    )
    in_specs = [
        pl.BlockSpec((blk, hid), lambda i: (i, 0)),
        pl.BlockSpec((_E, hid), lambda i: (0, 0)),
    ]
    out_specs = (
        pl.BlockSpec((blk, _TOP_K), lambda i: (i, 0)),
        pl.BlockSpec((blk, _TOP_K), lambda i: (i, 0)),
        pl.BlockSpec((bsz, _E), lambda i: (0, 0)),
        pl.BlockSpec((bsz, _E), lambda i: (0, 0)),
        pl.BlockSpec((1, 1), lambda i: (0, 0)),
    )
    idx, wt, _ce, _ss, aux = pl.pallas_call(
        functools.partial(
            _router_kernel, blk=blk, nsteps=nsteps,
            blocks_per_batch=seq_len // blk, bsz=bsz, seq_len=seq_len),
        grid=(nsteps,),
        in_specs=in_specs,
        out_specs=out_specs,
        out_shape=out_shapes,
        compiler_params=pltpu.CompilerParams(
            dimension_semantics=("arbitrary",)),
    )(hs, weight)
    idx = idx.reshape(n_tok, _TOP_K)
    wt = wt.reshape(n_tok, _TOP_K)
    return idx, wt, aux[0, 0]


# speedup vs baseline: 3.9246x; 1.2583x over previous
import functools

import jax
import jax.numpy as jnp
from jax import lax
from jax.experimental import pallas as pl
from jax.experimental.pallas import tpu as pltpu

_TOP_K = 8
_E = 64
_ALPHA = 0.1


def _router_kernel(hs_ref, w_ref, idx_ref, wt_ref, ce_ref, ss_ref, aux_ref,
                   *, blk, nsteps, blocks_per_batch, bsz, seq_len):
    i = pl.program_id(0)

    @pl.when(i == 0)
    def _init():
        ce_ref[:, :] = jnp.zeros_like(ce_ref)
        ss_ref[:, :] = jnp.zeros_like(ss_ref)

    logits = lax.dot_general(
        w_ref[:, :], hs_ref[:, :], (((1,), (1,)), ((), ())),
        preferred_element_type=jnp.float32,
        precision=lax.Precision.DEFAULT)

    m = jnp.max(logits, axis=0, keepdims=True)
    e = jnp.exp(logits - m)

    rows = lax.broadcasted_iota(jnp.int32, (_E, blk), 0)
    work = e
    vals, idxs = [], []
    for _ in range(_TOP_K):
        mx = jnp.max(work, axis=0, keepdims=True)
        pick = jnp.argmax(work, axis=0).reshape(1, blk).astype(jnp.int32)
        vals.append(mx)
        idxs.append(pick)
        work = jnp.where(rows == pick, -jnp.inf, work)

    topw = jnp.concatenate(vals, axis=0)
    topi = jnp.concatenate(idxs, axis=0)
    denom = jnp.sum(topw, axis=0, keepdims=True) + 1e-20
    wt_ref[:, :] = topw / denom
    idx_ref[:, :] = topi

    sel = (work == -jnp.inf).astype(jnp.float32)
    ones_row = jnp.ones((1, blk), jnp.float32)
    counts = lax.dot_general(
        sel, ones_row, (((1,), (1,)), ((), ())),
        preferred_element_type=jnp.float32)
    s = jnp.sum(e, axis=0, keepdims=True)
    recip_s = (1.0 / s)
    ssum = lax.dot_general(
        e, recip_s, (((1,), (1,)), ((), ())),
        preferred_element_type=jnp.float32)

    b = i // blocks_per_batch
    bcol = lax.broadcasted_iota(jnp.int32, (1, bsz), 1)
    bmask = (bcol == b).astype(jnp.float32)
    ce_ref[:, :] += counts * bmask
    ss_ref[:, :] += ssum * bmask

    @pl.when(i == nsteps - 1)
    def _fin():
        ce = ce_ref[:, :] * (_E / (seq_len * _TOP_K))
        ms = ss_ref[:, :] / seq_len
        aux_ref[:, :] = jnp.sum(ce * ms, keepdims=True).reshape(1, 1) * (_ALPHA / bsz)


def kernel(hidden_states, weight):
    bsz, seq_len, hid = hidden_states.shape
    n_tok = bsz * seq_len
    blk = 1024
    nsteps = n_tok // blk
    hs = hidden_states.reshape(n_tok, hid)

    out_shapes = (
        jax.ShapeDtypeStruct((_TOP_K, n_tok), jnp.int32),
        jax.ShapeDtypeStruct((_TOP_K, n_tok), jnp.float32),
        jax.ShapeDtypeStruct((_E, bsz), jnp.float32),
        jax.ShapeDtypeStruct((_E, bsz), jnp.float32),
        jax.ShapeDtypeStruct((1, 1), jnp.float32),
    )
    in_specs = [
        pl.BlockSpec((blk, hid), lambda i: (i, 0)),
        pl.BlockSpec((_E, hid), lambda i: (0, 0)),
    ]
    out_specs = (
        pl.BlockSpec((_TOP_K, blk), lambda i: (0, i)),
        pl.BlockSpec((_TOP_K, blk), lambda i: (0, i)),
        pl.BlockSpec((_E, bsz), lambda i: (0, 0)),
        pl.BlockSpec((_E, bsz), lambda i: (0, 0)),
        pl.BlockSpec((1, 1), lambda i: (0, 0)),
    )
    idx_t, wt_t, _ce, _ss, aux = pl.pallas_call(
        functools.partial(
            _router_kernel, blk=blk, nsteps=nsteps,
            blocks_per_batch=seq_len // blk, bsz=bsz, seq_len=seq_len),
        grid=(nsteps,),
        in_specs=in_specs,
        out_specs=out_specs,
        out_shape=out_shapes,
        compiler_params=pltpu.CompilerParams(
            dimension_semantics=("arbitrary",)),
    )(hs, weight)
    return idx_t.T, wt_t.T, aux[0, 0]
